# Initial kernel scaffold; baseline (speedup 1.0000x reference)
import jax
import jax.numpy as jnp
from jax.experimental import pallas as pl


def kernel(x, edge_index, edge_weights, mlp_W, mlp_b, att, Wz, bz, Wr, br, Wh, bh, lzW, lzb, lrW, lrb, lhW, lhb, lin_W, lin_b):
    raise NotImplementedError("write your pallas kernel here")


# speedup vs baseline: 11.4113x; 11.4113x over previous
import functools

import jax
import jax.numpy as jnp
from jax import lax
from jax.experimental import pallas as pl
from jax.experimental.pallas import tpu as pltpu
from jax.experimental.pallas import tpu_sc as plsc

N = 10000
F = 128
T = 12
OUT = 32
GW = T * 64
NPAD = 10240
ROWS = 128
NOWNERS = NPAD // ROWS
NPASS = 3
E = 320000
EPAD = 327680
ECHUNK = 2048
EROWS_PER_TILE = (EPAD // 128) // 32
WCAP = 2048
GB = 16
NW = 32


def _mesh():
    return plsc.VectorSubcoreMesh(
        core_axis_name="c", subcore_axis_name="s", num_cores=2,
        num_subcores=16)


@functools.partial(
    pl.kernel,
    out_type=jax.ShapeDtypeStruct((2, NPAD), jnp.float32),
    scratch_types=[
        pltpu.VMEM((EROWS_PER_TILE, 128), jnp.int32),
        pltpu.VMEM((EROWS_PER_TILE, 128), jnp.float32),
        pltpu.VMEM((NPAD,), jnp.float32),
        pltpu.VMEM_SHARED((NPAD,), jnp.float32),
    ],
    mesh=_mesh(),
)
def _deg_kernel(dst2d, w2d, out, dstb, wb, zbuf, degsh):
    c = lax.axis_index("c")
    s = lax.axis_index("s")
    wid = s * 2 + c

    @pl.when(s == 0)
    def _zero():
        def zb(i, carry):
            zbuf[pl.ds(i * 16, 16)] = jnp.zeros((16,), jnp.float32)
            return carry
        lax.fori_loop(0, NPAD // 16, zb, 0)
        pltpu.sync_copy(zbuf, degsh)

    plsc.subcore_barrier()

    pltpu.sync_copy(dst2d.at[pl.ds(wid * EROWS_PER_TILE, EROWS_PER_TILE)], dstb)
    pltpu.sync_copy(w2d.at[pl.ds(wid * EROWS_PER_TILE, EROWS_PER_TILE)], wb)

    def body(j, carry):
        pltpu.sync_copy(wb.at[j], degsh.at[dstb.at[j]], add=True)
        return carry
    lax.fori_loop(0, EROWS_PER_TILE, body, 0)

    plsc.subcore_barrier()

    @pl.when(s == 0)
    def _writeback():
        pltpu.sync_copy(degsh, out.at[c])


def _proj_body(xt_ref, mlp3_ref, mlpb_ref, degT_ref, Wz_ref, Wh_ref,
               lzW_ref, lhW_ref, gp_ref):
    X = xt_ref[...]
    mlp3 = mlp3_ref[...]
    dn = (((1,), (1,)), ((), ()))

    deg = jnp.sum(degT_ref[...], axis=1, keepdims=True) + 1.0
    dinv = lax.rsqrt(deg)

    acc = jnp.zeros((X.shape[1], T), jnp.float32)
    for t in range(T):
        acc = acc + lax.dot_general(X[t], mlp3[:, :, t], dn,
                                    preferred_element_type=jnp.float32)
    fw = jax.nn.sigmoid(acc + mlpb_ref[...])

    W2 = jnp.concatenate(
        [lax.dot_general(Wz_ref[...], lzW_ref[...][:, :OUT], dn,
                         preferred_element_type=jnp.float32),
         lax.dot_general(Wh_ref[...], lhW_ref[...][:, :OUT], dn,
                         preferred_element_type=jnp.float32)], axis=1)

    for t in range(T):
        P = jnp.dot(X[t], W2, preferred_element_type=jnp.float32)
        gp_ref[:, t * 64:(t + 1) * 64] = P * (dinv * fw[:, t:t + 1])


@functools.partial(
    pl.kernel,
    out_type=jax.ShapeDtypeStruct((NPAD, GW), jnp.float32),
    scratch_types=[
        pltpu.VMEM((ECHUNK,), jnp.int32),
        pltpu.VMEM((ECHUNK,), jnp.int32),
        pltpu.VMEM((ECHUNK,), jnp.float32),
        pltpu.VMEM((WCAP,), jnp.int32),
        pltpu.VMEM((WCAP,), jnp.int32),
        pltpu.VMEM((WCAP,), jnp.float32),
        pltpu.VMEM((GB, GW), jnp.float32),
        pltpu.VMEM((ROWS, GW), jnp.float32),
        pltpu.VMEM((16,), jnp.int32),
        pltpu.SMEM((16,), jnp.int32),
        pltpu.SemaphoreType.DMA,
    ],
    mesh=_mesh(),
    compiler_params=pltpu.CompilerParams(needs_layout_passes=False),
)
def _spmm_kernel(srcp, dstp, wp, gp, out, srcb, dstb, wb,
                 wls, wll, wlw, stage, accum, v16, m16, sem):
    c = lax.axis_index("c")
    s = lax.axis_index("s")
    wid = s * 2 + c
    iota = lax.iota(jnp.int32, 16)
    i15 = jnp.full((16,), 15, jnp.int32)

    def zwl(i, carry):
        wls[pl.ds(i * 16, 16)] = jnp.zeros((16,), jnp.int32)
        return carry
    lax.fori_loop(0, WCAP // 16, zwl, 0)

    def drain(m):
        def dbody(b, carry):
            k0 = b * GB
            idx = wls.at[pl.ds(k0, GB)]
            pltpu.async_copy(gp.at[idx], stage, sem).wait()
            cnt_b = jnp.minimum(m - k0, GB)

            def ebody(k, ecarry):
                ksp = jnp.full((16,), k0 + k, jnp.int32)
                row = plsc.load_gather(wll, [ksp])
                wsp = plsc.load_gather(wlw, [ksp])
                for j in range(GW // 16):
                    v = stage[k, pl.ds(j * 16, 16)]
                    plsc.addupdate_scatter(
                        accum, [row, iota + (j * 16)], v * wsp)
                return ecarry
            lax.fori_loop(0, cnt_b, ebody, 0)
            return carry
        lax.fori_loop(0, (m + GB - 1) // GB, dbody, 0)

    for p in range(NPASS):
        owner = p * 32 + wid
        base = owner * ROWS

        @pl.when(owner < NOWNERS)
        def _pass():
            pltpu.sync_copy(gp.at[pl.ds(base, ROWS)], accum)

            def chunk(ci, carry):
                pltpu.sync_copy(srcp.at[pl.ds(ci * ECHUNK, ECHUNK)], srcb)
                pltpu.sync_copy(dstp.at[pl.ds(ci * ECHUNK, ECHUNK)], dstb)
                pltpu.sync_copy(wp.at[pl.ds(ci * ECHUNK, ECHUNK)], wb)

                def vreg(i, off2):
                    d16 = dstb[pl.ds(i * 16, 16)]
                    local = d16 - base
                    mask = (local >= 0) & (local < ROWS)
                    mi = mask.astype(jnp.int32)
                    csum = plsc.cumsum(mi)
                    pos = csum - mi + off2
                    v16[pl.ds(0, 16)] = csum
                    tot = plsc.load_gather(v16, [i15])
                    s16 = srcb[pl.ds(i * 16, 16)]
                    w16 = wb[pl.ds(i * 16, 16)]
                    plsc.store_scatter(wls, [pos], s16, mask=mask)
                    plsc.store_scatter(wll, [pos], local, mask=mask)
                    plsc.store_scatter(wlw, [pos], w16, mask=mask)
                    return off2 + tot

                off = lax.fori_loop(0, ECHUNK // 16, vreg,
                                    jnp.zeros((16,), jnp.int32))
                m = off[0]

                @pl.when(m > 0)
                def _dr():
                    drain(m)
                return carry

            lax.fori_loop(0, EPAD // ECHUNK, chunk, 0)
            pltpu.sync_copy(accum, out.at[pl.ds(base, ROWS)])


def _post_body(u_ref, degT_ref, att_ref, bz_ref, bh_ref, lzb_ref, lhb_ref,
               lzW_ref, lhW_ref, linW_ref, linb_ref, y_ref):
    dn = (((1,), (1,)), ((), ()))
    deg = jnp.sum(degT_ref[...], axis=1, keepdims=True) + 1.0
    dinv = lax.rsqrt(deg)
    S = u_ref[...] * dinv

    bz2 = lax.dot_general(bz_ref[...], lzW_ref[...][:, :OUT], dn,
                          preferred_element_type=jnp.float32) + lzb_ref[...]
    bh2 = lax.dot_general(bh_ref[...], lhW_ref[...][:, :OUT], dn,
                          preferred_element_type=jnp.float32) + lhb_ref[...]

    probs = jax.nn.softmax(att_ref[...], axis=1)

    H = jnp.zeros((S.shape[0], OUT), jnp.float32)
    for t in range(T):
        Z = jax.nn.sigmoid(S[:, t * 64:t * 64 + OUT] + bz2)
        Htl = jnp.tanh(S[:, t * 64 + OUT:(t + 1) * 64] + bh2)
        H = H + probs[0:1, t:t + 1] * (1.0 - Z) * Htl

    y = lax.dot_general(jax.nn.relu(H), linW_ref[...], dn,
                        preferred_element_type=jnp.float32) + linb_ref[...]
    y_ref[...] = y


def kernel(x, edge_index, edge_weights, mlp_W, mlp_b, att, Wz, bz, Wr, br,
           Wh, bh, lzW, lzb, lrW, lrb, lhW, lhb, lin_W, lin_b):
    del Wr, br, lrW, lrb

    src = edge_index[0]
    dst = edge_index[1]
    npad = EPAD - E

    dst_a = jnp.concatenate(
        [dst, jnp.full((npad,), NPAD - 1, jnp.int32)]).reshape(-1, 128)
    w_a = jnp.concatenate(
        [edge_weights, jnp.zeros((npad,), jnp.float32)]).reshape(-1, 128)
    deg2 = _deg_kernel(dst_a, w_a)
    degT = deg2.T

    xt = x.transpose(2, 0, 1)
    mlp3 = mlp_W.reshape(T, F, T)
    BN = 256
    grid = (NPAD // BN,)
    gp = pl.pallas_call(
        _proj_body,
        grid=grid,
        in_specs=[
            pl.BlockSpec((T, BN, F), lambda i: (0, i, 0)),
            pl.BlockSpec((T, F, T), lambda i: (0, 0, 0)),
            pl.BlockSpec((1, T), lambda i: (0, 0)),
            pl.BlockSpec((BN, 2), lambda i: (i, 0)),
            pl.BlockSpec((F, OUT), lambda i: (0, 0)),
            pl.BlockSpec((F, OUT), lambda i: (0, 0)),
            pl.BlockSpec((OUT, 2 * OUT), lambda i: (0, 0)),
            pl.BlockSpec((OUT, 2 * OUT), lambda i: (0, 0)),
        ],
        out_specs=pl.BlockSpec((BN, GW), lambda i: (i, 0)),
        out_shape=jax.ShapeDtypeStruct((NPAD, GW), jnp.float32),
    )(xt, mlp3, mlp_b.reshape(1, T), degT, Wz, Wh, lzW, lhW)

    src_c = jnp.concatenate([src, jnp.zeros((npad,), jnp.int32)])
    dst_c = jnp.concatenate([dst, jnp.full((npad,), jnp.int32(1 << 20))])
    w_c = jnp.concatenate([edge_weights, jnp.zeros((npad,), jnp.float32)])
    u = _spmm_kernel(src_c, dst_c, w_c, gp)

    y = pl.pallas_call(
        _post_body,
        grid=grid,
        in_specs=[
            pl.BlockSpec((BN, GW), lambda i: (i, 0)),
            pl.BlockSpec((BN, 2), lambda i: (i, 0)),
            pl.BlockSpec((1, T), lambda i: (0, 0)),
            pl.BlockSpec((1, OUT), lambda i: (0, 0)),
            pl.BlockSpec((1, OUT), lambda i: (0, 0)),
            pl.BlockSpec((1, OUT), lambda i: (0, 0)),
            pl.BlockSpec((1, OUT), lambda i: (0, 0)),
            pl.BlockSpec((OUT, 2 * OUT), lambda i: (0, 0)),
            pl.BlockSpec((OUT, 2 * OUT), lambda i: (0, 0)),
            pl.BlockSpec((T, OUT), lambda i: (0, 0)),
            pl.BlockSpec((1, T), lambda i: (0, 0)),
        ],
        out_specs=pl.BlockSpec((BN, T), lambda i: (i, 0)),
        out_shape=jax.ShapeDtypeStruct((NPAD, T), jnp.float32),
    )(u, degT, att.reshape(1, T), bz.reshape(1, OUT), bh.reshape(1, OUT),
      lzb.reshape(1, OUT), lhb.reshape(1, OUT), lzW, lhW, lin_W,
      lin_b.reshape(1, T))

    return y[:N]


# speedup vs baseline: 13.9967x; 1.2266x over previous
import functools

import jax
import jax.numpy as jnp
from jax import lax
from jax.experimental import pallas as pl
from jax.experimental.pallas import tpu as pltpu
from jax.experimental.pallas import tpu_sc as plsc

N = 10000
F = 128
T = 12
OUT = 32
GW = T * 64
NPAD = 10240
ROWS = 128
NOWNERS = NPAD // ROWS
NPASS = 3
E = 320000
EPAD = 327680
ECHUNK = 1024
NCHUNK = EPAD // ECHUNK
EROWS_PER_TILE = (EPAD // 128) // 32
WCAP = ECHUNK
GB = 8
NW = 32


def _mesh():
    return plsc.VectorSubcoreMesh(
        core_axis_name="c", subcore_axis_name="s", num_cores=2,
        num_subcores=16)


@functools.partial(
    pl.kernel,
    out_type=jax.ShapeDtypeStruct((2, NPAD), jnp.float32),
    scratch_types=[
        pltpu.VMEM((EROWS_PER_TILE, 128), jnp.int32),
        pltpu.VMEM((EROWS_PER_TILE, 128), jnp.float32),
        pltpu.VMEM((NPAD,), jnp.float32),
        pltpu.VMEM_SHARED((NPAD,), jnp.float32),
    ],
    mesh=_mesh(),
)
def _deg_kernel(dst2d, w2d, out, dstb, wb, zbuf, degsh):
    c = lax.axis_index("c")
    s = lax.axis_index("s")
    wid = s * 2 + c

    @pl.when(s == 0)
    def _zero():
        def zb(i, carry):
            zbuf[pl.ds(i * 16, 16)] = jnp.zeros((16,), jnp.float32)
            return carry
        lax.fori_loop(0, NPAD // 16, zb, 0)
        pltpu.sync_copy(zbuf, degsh)

    plsc.subcore_barrier()

    pltpu.sync_copy(dst2d.at[pl.ds(wid * EROWS_PER_TILE, EROWS_PER_TILE)], dstb)
    pltpu.sync_copy(w2d.at[pl.ds(wid * EROWS_PER_TILE, EROWS_PER_TILE)], wb)

    def body(j, carry):
        pltpu.sync_copy(wb.at[j], degsh.at[dstb.at[j]], add=True)
        return carry
    lax.fori_loop(0, EROWS_PER_TILE, body, 0)

    plsc.subcore_barrier()

    @pl.when(s == 0)
    def _writeback():
        pltpu.sync_copy(degsh, out.at[c])


def _proj_body(xt_ref, mlp3_ref, mlpb_ref, degT_ref, Wz_ref, Wh_ref,
               lzW_ref, lhW_ref, gp_ref):
    X = xt_ref[...]
    mlp3 = mlp3_ref[...]
    dn = (((1,), (1,)), ((), ()))

    deg = jnp.sum(degT_ref[...], axis=1, keepdims=True) + 1.0
    dinv = lax.rsqrt(deg)

    acc = jnp.zeros((X.shape[1], T), jnp.float32)
    for t in range(T):
        acc = acc + lax.dot_general(X[t], mlp3[:, :, t], dn,
                                    preferred_element_type=jnp.float32)
    fw = jax.nn.sigmoid(acc + mlpb_ref[...])

    W2 = jnp.concatenate(
        [lax.dot_general(Wz_ref[...], lzW_ref[...][:, :OUT], dn,
                         preferred_element_type=jnp.float32),
         lax.dot_general(Wh_ref[...], lhW_ref[...][:, :OUT], dn,
                         preferred_element_type=jnp.float32)], axis=1)

    for t in range(T):
        P = jnp.dot(X[t], W2, preferred_element_type=jnp.float32)
        gp_ref[:, t * 64:(t + 1) * 64] = P * (dinv * fw[:, t:t + 1])


@functools.partial(
    pl.kernel,
    out_type=jax.ShapeDtypeStruct((NPAD, GW), jnp.float32),
    scratch_types=[
        pltpu.VMEM((2, ECHUNK), jnp.int32),
        pltpu.VMEM((2, ECHUNK), jnp.int32),
        pltpu.VMEM((2, ECHUNK), jnp.float32),
        pltpu.VMEM((WCAP,), jnp.int32),
        pltpu.VMEM((WCAP,), jnp.int32),
        pltpu.VMEM((WCAP,), jnp.float32),
        pltpu.VMEM((2, GB, GW), jnp.float32),
        pltpu.VMEM((ROWS, GW), jnp.float32),
        pltpu.VMEM((16,), jnp.int32),
        pltpu.SemaphoreType.DMA,
        pltpu.SemaphoreType.DMA,
        pltpu.SemaphoreType.DMA,
    ],
    mesh=_mesh(),
    compiler_params=pltpu.CompilerParams(needs_layout_passes=False),
)
def _spmm_kernel(srcp, dstp, wp, gp, out, srcb, dstb, wb,
                 wls, wll, wlw, stage, accum, v16, semc0, semc1, semg):
    c = lax.axis_index("c")
    s = lax.axis_index("s")
    wid = s * 2 + c
    iota = lax.iota(jnp.int32, 16)
    i15 = jnp.full((16,), 15, jnp.int32)
    semc = [semc0, semc1]

    def zwl(i, carry):
        wls[pl.ds(i * 16, 16)] = jnp.zeros((16,), jnp.int32)
        return carry
    lax.fori_loop(0, WCAP // 16, zwl, 0)

    def cstart(ci, b):
        pltpu.async_copy(srcp.at[pl.ds(ci * ECHUNK, ECHUNK)],
                         srcb.at[b], semc[b])
        pltpu.async_copy(dstp.at[pl.ds(ci * ECHUNK, ECHUNK)],
                         dstb.at[b], semc[b])
        pltpu.async_copy(wp.at[pl.ds(ci * ECHUNK, ECHUNK)],
                         wb.at[b], semc[b])

    def cwait(b):
        pltpu.make_async_copy(srcp.at[pl.ds(0, ECHUNK)],
                              srcb.at[b], semc[b]).wait()
        pltpu.make_async_copy(dstp.at[pl.ds(0, ECHUNK)],
                              dstb.at[b], semc[b]).wait()
        pltpu.make_async_copy(wp.at[pl.ds(0, ECHUNK)],
                              wb.at[b], semc[b]).wait()

    def gstart(bidx, sb):
        pltpu.async_copy(gp.at[wls.at[pl.ds(bidx * GB, GB)]],
                         stage.at[sb], semg)

    def gwait(sb):
        pltpu.make_async_copy(gp.at[pl.ds(0, GB)],
                              stage.at[sb], semg).wait()

    def drain(m):
        nb = (m + GB - 1) // GB
        gstart(0, 0)

        @pl.when(nb > 1)
        def _g1():
            gstart(1, 1)

        def pair(pb, carry):
            for sb in range(2):
                bidx = pb * 2 + sb

                @pl.when(bidx < nb)
                def _do():
                    gwait(sb)
                    k0 = bidx * GB
                    cnt_b = jnp.minimum(m - k0, GB)

                    def ebody(k, ecarry):
                        ksp = jnp.full((16,), k0 + k, jnp.int32)
                        row = plsc.load_gather(wll, [ksp])
                        wsp = plsc.load_gather(wlw, [ksp])
                        for j in range(GW // 16):
                            v = stage[sb, k, pl.ds(j * 16, 16)]
                            plsc.addupdate_scatter(
                                accum, [row, iota + (j * 16)], v * wsp)
                        return ecarry
                    lax.fori_loop(0, cnt_b, ebody, 0)

                    @pl.when(bidx + 2 < nb)
                    def _nxt():
                        gstart(bidx + 2, sb)
            return carry
        lax.fori_loop(0, (nb + 1) // 2, pair, 0)

    for p in range(NPASS):
        owner = p * 32 + wid
        base = owner * ROWS

        @pl.when(owner < NOWNERS)
        def _pass():
            pltpu.sync_copy(gp.at[pl.ds(base, ROWS)], accum)
            cstart(0, 0)
            cstart(1, 1)

            def cpair(ck, carry):
                for b in range(2):
                    ci = ck * 2 + b
                    cwait(b)

                    def vreg(i, off2):
                        d16 = dstb[b, pl.ds(i * 16, 16)]
                        local = d16 - base
                        mask = (local >= 0) & (local < ROWS)
                        mi = mask.astype(jnp.int32)
                        csum = plsc.cumsum(mi)
                        pos = csum - mi + off2
                        v16[pl.ds(0, 16)] = csum
                        tot = plsc.load_gather(v16, [i15])
                        s16 = srcb[b, pl.ds(i * 16, 16)]
                        w16 = wb[b, pl.ds(i * 16, 16)]
                        plsc.store_scatter(wls, [pos], s16, mask=mask)
                        plsc.store_scatter(wll, [pos], local, mask=mask)
                        plsc.store_scatter(wlw, [pos], w16, mask=mask)
                        return off2 + tot

                    off = lax.fori_loop(0, ECHUNK // 16, vreg,
                                        jnp.zeros((16,), jnp.int32))

                    @pl.when(ci + 2 < NCHUNK)
                    def _pref():
                        cstart(ci + 2, b)

                    m = off[0]

                    @pl.when(m > 0)
                    def _dr():
                        drain(m)
                return carry

            lax.fori_loop(0, NCHUNK // 2, cpair, 0)
            pltpu.sync_copy(accum, out.at[pl.ds(base, ROWS)])


def _post_body(u_ref, degT_ref, att_ref, bz_ref, bh_ref, lzb_ref, lhb_ref,
               lzW_ref, lhW_ref, linW_ref, linb_ref, y_ref):
    dn = (((1,), (1,)), ((), ()))
    deg = jnp.sum(degT_ref[...], axis=1, keepdims=True) + 1.0
    dinv = lax.rsqrt(deg)
    S = u_ref[...] * dinv

    bz2 = lax.dot_general(bz_ref[...], lzW_ref[...][:, :OUT], dn,
                          preferred_element_type=jnp.float32) + lzb_ref[...]
    bh2 = lax.dot_general(bh_ref[...], lhW_ref[...][:, :OUT], dn,
                          preferred_element_type=jnp.float32) + lhb_ref[...]

    probs = jax.nn.softmax(att_ref[...], axis=1)

    H = jnp.zeros((S.shape[0], OUT), jnp.float32)
    for t in range(T):
        Z = jax.nn.sigmoid(S[:, t * 64:t * 64 + OUT] + bz2)
        Htl = jnp.tanh(S[:, t * 64 + OUT:(t + 1) * 64] + bh2)
        H = H + probs[0:1, t:t + 1] * (1.0 - Z) * Htl

    y = lax.dot_general(jax.nn.relu(H), linW_ref[...], dn,
                        preferred_element_type=jnp.float32) + linb_ref[...]
    y_ref[...] = y


def kernel(x, edge_index, edge_weights, mlp_W, mlp_b, att, Wz, bz, Wr, br,
           Wh, bh, lzW, lzb, lrW, lrb, lhW, lhb, lin_W, lin_b):
    del Wr, br, lrW, lrb

    src = edge_index[0]
    dst = edge_index[1]
    npad = EPAD - E

    dst_a = jnp.concatenate(
        [dst, jnp.full((npad,), NPAD - 1, jnp.int32)]).reshape(-1, 128)
    w_a = jnp.concatenate(
        [edge_weights, jnp.zeros((npad,), jnp.float32)]).reshape(-1, 128)
    deg2 = _deg_kernel(dst_a, w_a)
    degT = deg2.T

    xt = x.transpose(2, 0, 1)
    mlp3 = mlp_W.reshape(T, F, T)
    BN = 256
    grid = (NPAD // BN,)
    gp = pl.pallas_call(
        _proj_body,
        grid=grid,
        in_specs=[
            pl.BlockSpec((T, BN, F), lambda i: (0, i, 0)),
            pl.BlockSpec((T, F, T), lambda i: (0, 0, 0)),
            pl.BlockSpec((1, T), lambda i: (0, 0)),
            pl.BlockSpec((BN, 2), lambda i: (i, 0)),
            pl.BlockSpec((F, OUT), lambda i: (0, 0)),
            pl.BlockSpec((F, OUT), lambda i: (0, 0)),
            pl.BlockSpec((OUT, 2 * OUT), lambda i: (0, 0)),
            pl.BlockSpec((OUT, 2 * OUT), lambda i: (0, 0)),
        ],
        out_specs=pl.BlockSpec((BN, GW), lambda i: (i, 0)),
        out_shape=jax.ShapeDtypeStruct((NPAD, GW), jnp.float32),
    )(xt, mlp3, mlp_b.reshape(1, T), degT, Wz, Wh, lzW, lhW)

    src_c = jnp.concatenate([src, jnp.zeros((npad,), jnp.int32)])
    dst_c = jnp.concatenate([dst, jnp.full((npad,), jnp.int32(1 << 20))])
    w_c = jnp.concatenate([edge_weights, jnp.zeros((npad,), jnp.float32)])
    u = _spmm_kernel(src_c, dst_c, w_c, gp)

    y = pl.pallas_call(
        _post_body,
        grid=grid,
        in_specs=[
            pl.BlockSpec((BN, GW), lambda i: (i, 0)),
            pl.BlockSpec((BN, 2), lambda i: (i, 0)),
            pl.BlockSpec((1, T), lambda i: (0, 0)),
            pl.BlockSpec((1, OUT), lambda i: (0, 0)),
            pl.BlockSpec((1, OUT), lambda i: (0, 0)),
            pl.BlockSpec((1, OUT), lambda i: (0, 0)),
            pl.BlockSpec((1, OUT), lambda i: (0, 0)),
            pl.BlockSpec((OUT, 2 * OUT), lambda i: (0, 0)),
            pl.BlockSpec((OUT, 2 * OUT), lambda i: (0, 0)),
            pl.BlockSpec((T, OUT), lambda i: (0, 0)),
            pl.BlockSpec((1, T), lambda i: (0, 0)),
        ],
        out_specs=pl.BlockSpec((BN, T), lambda i: (i, 0)),
        out_shape=jax.ShapeDtypeStruct((NPAD, T), jnp.float32),
    )(u, degT, att.reshape(1, T), bz.reshape(1, OUT), bh.reshape(1, OUT),
      lzb.reshape(1, OUT), lhb.reshape(1, OUT), lzW, lhW, lin_W,
      lin_b.reshape(1, T))

    return y[:N]
